# Initial kernel scaffold; baseline (speedup 1.0000x reference)
import jax
import jax.numpy as jnp
from jax.experimental import pallas as pl


def kernel(x, weight):
    raise NotImplementedError("write your pallas kernel here")


# speedup vs baseline: 3.6869x; 3.6869x over previous
import functools

import jax
import jax.numpy as jnp
from jax import lax
from jax.experimental import pallas as pl
from jax.experimental.pallas import tpu as pltpu
from jax.experimental.pallas import tpu_sc as plsc

_SCALE = 10.0
_NUM_ROWS = 100000
_DIM = 64
_BATCH = 16384 * 50
_CHUNK = 128
_NC = 2
_NS = 16
_NW = _NC * _NS
_CHUNKS_TOTAL = _BATCH // _CHUNK
_CHUNKS_PER_W = _CHUNKS_TOTAL // _NW


def _sc_body(w_hbm, idx_hbm, out_hbm, idx_v, rows_v, sem):
    wid = lax.axis_index("s") * _NC + lax.axis_index("c")
    row_base = wid * _CHUNKS_PER_W
    pltpu.sync_copy(idx_hbm.at[pl.ds(row_base, _CHUNKS_PER_W)], idx_v)

    def chunk(j, carry):
        pltpu.async_copy(w_hbm.at[idx_v.at[j]], rows_v, sem).wait()

        def scale_row(r, c2):
            for c in range(_DIM // 16):
                sl = pl.ds(c * 16, 16)
                rows_v[r, sl] = rows_v[r, sl] * _SCALE
            return c2

        lax.fori_loop(0, _CHUNK, scale_row, 0, unroll=4)

        pltpu.sync_copy(rows_v, out_hbm.at[pl.ds((row_base + j) * _CHUNK, _CHUNK)])
        return carry

    lax.fori_loop(0, _CHUNKS_PER_W, chunk, 0)


@functools.partial(jax.jit, static_argnames=())
def kernel(x, weight):
    idx = x.reshape(-1).astype(jnp.int32).reshape(_CHUNKS_TOTAL, _CHUNK)
    mesh = plsc.VectorSubcoreMesh(core_axis_name="c", subcore_axis_name="s")
    out = pl.kernel(
        _sc_body,
        mesh=mesh,
        out_type=jax.ShapeDtypeStruct((_BATCH, _DIM), jnp.float32),
        scratch_types=[
            pltpu.VMEM((_CHUNKS_PER_W, _CHUNK), jnp.int32),
            pltpu.VMEM((_CHUNK, _DIM), jnp.float32),
            pltpu.SemaphoreType.DMA,
        ],
        compiler_params=pltpu.CompilerParams(use_tc_tiling_on_sc=False),
    )(weight, idx)
    return out.reshape(x.shape[0], x.shape[1], _DIM)


# speedup vs baseline: 4.7071x; 1.2767x over previous
import functools

import jax
import jax.numpy as jnp
from jax import lax
from jax.experimental import pallas as pl
from jax.experimental.pallas import tpu as pltpu
from jax.experimental.pallas import tpu_sc as plsc

_SCALE = 10.0
_DIM = 64
_BATCH = 16384 * 50
_CHUNK = 128
_NC = 2
_NS = 16
_NW = _NC * _NS
_CHUNKS_TOTAL = _BATCH // _CHUNK
_CHUNKS_PER_W = _CHUNKS_TOTAL // _NW
_SUP_CHUNKS = 2
_SUP = _SUP_CHUNKS * _CHUNK
_SUPERS = _CHUNKS_PER_W // _SUP_CHUNKS
_NB = 4
_LOOKAHEAD = 2


def _sc_body(w_hbm, idx_hbm, out_hbm, idx_v, g0, g1, g2, g3, *sems):
    gs = sems[:_NB]
    os_ = sems[_NB:]
    bufs = (g0, g1, g2, g3)
    wid = lax.axis_index("s") * _NC + lax.axis_index("c")
    row_base = wid * _CHUNKS_PER_W
    out_base = row_base * _CHUNK

    pltpu.sync_copy(idx_hbm.at[pl.ds(row_base, _CHUNKS_PER_W)], idx_v)

    def gather(s, b, q):
        return pltpu.make_async_copy(
            w_hbm.at[idx_v.at[s * _SUP_CHUNKS + q]],
            bufs[b].at[pl.ds(q * _CHUNK, _CHUNK)],
            gs[b],
        )

    def out_copy(s, b):
        return pltpu.make_async_copy(
            bufs[b], out_hbm.at[pl.ds(out_base + s * _SUP, _SUP)], os_[b]
        )

    def fire(s, b):
        for q in range(_SUP_CHUNKS):
            gather(s, b, q).start()

    for b in range(_LOOKAHEAD):
        fire(b, b)

    def sup_iter(g, i):
        s = g * _NB + i
        b = i
        b2 = (i + _LOOKAHEAD) % _NB
        for q in range(_SUP_CHUNKS):
            gather(s, b, q).wait()

        @pl.when(s + _LOOKAHEAD < _SUPERS)
        def _():
            @pl.when(s >= _LOOKAHEAD)
            def _():
                out_copy(s - _NB + _LOOKAHEAD, b2).wait()

            fire(s + _LOOKAHEAD, b2)

        def scale_row(r, c2):
            for c in range(_DIM // 16):
                sl = pl.ds(c * 16, 16)
                bufs[b][r, sl] = bufs[b][r, sl] * _SCALE
            return c2

        lax.fori_loop(0, _SUP, scale_row, 0, unroll=8)

        out_copy(s, b).start()

    def outer(g, carry):
        for i in range(_NB):
            sup_iter(g, i)
        return carry

    lax.fori_loop(0, _SUPERS // _NB, outer, 0)

    for b in range(_NB):
        out_copy(_SUPERS - _NB + b, b).wait()


@functools.partial(jax.jit, static_argnames=())
def kernel(x, weight):
    idx = x.reshape(-1).astype(jnp.int32).reshape(_CHUNKS_TOTAL, _CHUNK)
    mesh = plsc.VectorSubcoreMesh(core_axis_name="c", subcore_axis_name="s")
    out = pl.kernel(
        _sc_body,
        mesh=mesh,
        out_type=jax.ShapeDtypeStruct((_BATCH, _DIM), jnp.float32),
        scratch_types=[
            pltpu.VMEM((_CHUNKS_PER_W, _CHUNK), jnp.int32),
        ]
        + [pltpu.VMEM((_SUP, _DIM), jnp.float32) for _ in range(_NB)]
        + [pltpu.SemaphoreType.DMA for _ in range(2 * _NB)],
        compiler_params=pltpu.CompilerParams(use_tc_tiling_on_sc=False),
    )(weight, idx)
    return out.reshape(x.shape[0], x.shape[1], _DIM)
